# baseline (speedup 1.0000x reference)
import functools

import jax
import jax.numpy as jnp
from jax.experimental import pallas as pl
from jax.experimental.pallas import tpu as pltpu

_NUM_QUERIES = 300
_EMBED_DIM = 256


def _make_body(batch):
    def _body(t_ref, o_ref, sem):
        copies = [pltpu.make_async_copy(t_ref, o_ref.at[b], sem) for b in range(batch)]
        for c in copies:
            c.start()
        for c in copies:
            c.wait()

    return _body


@functools.lru_cache(maxsize=None)
def _build(batch: int):
    return pl.pallas_call(
        _make_body(batch),
        in_specs=[pl.BlockSpec(memory_space=pltpu.VMEM)],
        out_specs=pl.BlockSpec(memory_space=pl.ANY),
        out_shape=jax.ShapeDtypeStruct((batch, _NUM_QUERIES, _EMBED_DIM), jnp.float32),
        scratch_shapes=[pltpu.SemaphoreType.DMA],
    )


def kernel(x, table):
    return _build(x.shape[0])(table)
